# baseline (speedup 1.0000x reference)
import functools

import jax
import jax.numpy as jnp
from jax import lax
from jax.experimental import pallas as pl
from jax.experimental.pallas import tpu as pltpu
from jax.experimental.pallas import tpu_sc as plsc

_B = 4096
_S = 8192
_SH = _S // 2
_D = 128
_BT = 2048
_L = 16

_info = plsc.get_sparse_core_info()
_NC = _info.num_cores
_NS = _info.num_subcores
_NW = _NC * _NS
_HPW = _SH // _NW
_BPW = _B // _NW

_sc_mesh = plsc.VectorSubcoreMesh(core_axis_name="c", subcore_axis_name="s")


@functools.partial(
    pl.kernel,
    mesh=_sc_mesh,
    out_type=jax.ShapeDtypeStruct((_SH, _D), jnp.float32),
    scratch_types=[
        pltpu.VMEM((_HPW,), jnp.int32),
        pltpu.VMEM((_HPW, _D), jnp.float32),
        pltpu.SemaphoreType.DMA,
    ],
)
def _sc_gather_chunk(weight_hbm, sids_hbm, out_s, sidx_v, srows_v, sem):
    wid = lax.axis_index("s") * _NC + lax.axis_index("c")
    sbase = wid * _HPW
    pltpu.sync_copy(sids_hbm.at[pl.ds(sbase, _HPW)], sidx_v)
    pltpu.async_copy(weight_hbm.at[sidx_v], srows_v, sem).wait()
    pltpu.sync_copy(srows_v, out_s.at[pl.ds(sbase, _HPW)])


@functools.partial(
    pl.kernel,
    mesh=_sc_mesh,
    out_type=jax.ShapeDtypeStruct((_NW, _L), jnp.float32),
    scratch_types=[
        pltpu.VMEM((_BPW,), jnp.int32),
        pltpu.VMEM((_BPW, _D), jnp.float32),
        pltpu.VMEM((_BPW, _D), jnp.float32),
        pltpu.VMEM((_L,), jnp.float32),
        pltpu.SemaphoreType.DMA,
    ],
)
def _sc_true_dot(x_hbm, labels_hbm, weight_hbm, out_p,
                 lidx_v, lrows_v, xrows_v, acc_v, sem):
    wid = lax.axis_index("s") * _NC + lax.axis_index("c")
    lbase = wid * _BPW
    pltpu.sync_copy(labels_hbm.at[pl.ds(lbase, _BPW)], lidx_v)
    cp_l = pltpu.async_copy(weight_hbm.at[lidx_v], lrows_v, sem)
    pltpu.sync_copy(x_hbm.at[pl.ds(lbase, _BPW)], xrows_v)
    cp_l.wait()

    def body(r, acc):
        for c in range(_D // _L):
            acc = acc + (lrows_v[r, pl.ds(c * _L, _L)]
                         * xrows_v[r, pl.ds(c * _L, _L)])
        return acc

    acc_v[...] = lax.fori_loop(0, _BPW, body, jnp.zeros((_L,), jnp.float32))
    pltpu.sync_copy(acc_v, out_p.at[wid])


def _rs1_body(x_ref, sw_ref, rs_ref):
    logits = lax.dot_general(
        x_ref[...], sw_ref[...], (((1,), (1,)), ((), ())),
        preferred_element_type=jnp.float32)
    rs_ref[...] = jnp.sum(jnp.exp(logits), axis=1, keepdims=True)


def _tc_rs1(x, sw1):
    return pl.pallas_call(
        _rs1_body,
        grid=(_B // _BT,),
        in_specs=[
            pl.BlockSpec((_BT, _D), lambda i: (i, 0)),
            pl.BlockSpec((_SH, _D), lambda i: (0, 0)),
        ],
        out_specs=pl.BlockSpec((_BT, 1), lambda i: (i, 0)),
        out_shape=jax.ShapeDtypeStruct((_B, 1), jnp.float32),
        cost_estimate=pl.CostEstimate(
            flops=2 * _B * _SH * _D, transcendentals=_B * _SH,
            bytes_accessed=(_B * _D * 4 + _SH * _D * 4)),
    )(x, sw1)


def _lse2_body(x_ref, sw_ref, rs1_ref, part_ref, out_ref):
    i = pl.program_id(0)
    logits = lax.dot_general(
        x_ref[...], sw_ref[...], (((1,), (1,)), ((), ())),
        preferred_element_type=jnp.float32)
    rowsum = jnp.sum(jnp.exp(logits), axis=1) + rs1_ref[..., 0]
    contrib = jnp.sum(jnp.log(rowsum))

    @pl.when(i == 0)
    def _():
        out_ref[0, 0] = contrib - jnp.sum(part_ref[...])

    @pl.when(i != 0)
    def _():
        out_ref[0, 0] += contrib


def _tc_lse2(x, sw2, rs1, part):
    out = pl.pallas_call(
        _lse2_body,
        grid=(_B // _BT,),
        in_specs=[
            pl.BlockSpec((_BT, _D), lambda i: (i, 0)),
            pl.BlockSpec((_SH, _D), lambda i: (0, 0)),
            pl.BlockSpec((_BT, 1), lambda i: (i, 0)),
            pl.BlockSpec((_NW, _L), lambda i: (0, 0)),
        ],
        out_specs=pl.BlockSpec((1, 1), lambda i: (0, 0),
                               memory_space=pltpu.SMEM),
        out_shape=jax.ShapeDtypeStruct((1, 1), jnp.float32),
        cost_estimate=pl.CostEstimate(
            flops=2 * _B * _SH * _D, transcendentals=_B * _SH,
            bytes_accessed=(_B * _D * 4 + _SH * _D * 4)),
    )(x, sw2, rs1, part)
    return out[0, 0]


def kernel(inputs, labels, sample_ids, weight):
    sids = sample_ids.astype(jnp.int32)
    sw1 = _sc_gather_chunk(weight, sids[:_SH])
    sw2 = _sc_gather_chunk(weight, sids[_SH:])
    part = _sc_true_dot(inputs, labels.astype(jnp.int32), weight)
    rs1 = _tc_rs1(inputs, sw1)
    return _tc_lse2(inputs, sw2, rs1, part)


# speedup vs baseline: 1.1190x; 1.1190x over previous
import functools

import jax
import jax.numpy as jnp
from jax import lax
from jax.experimental import pallas as pl
from jax.experimental.pallas import tpu as pltpu
from jax.experimental.pallas import tpu_sc as plsc

_B = 4096
_S = 8192
_D = 128
_BT = 2048
_L = 16

_info = plsc.get_sparse_core_info()
_NC = _info.num_cores
_NS = _info.num_subcores
_NW = _NC * _NS
_SPW = _S // _NW
_BPW = _B // _NW

_sc_mesh = plsc.VectorSubcoreMesh(core_axis_name="c", subcore_axis_name="s")


@functools.partial(
    pl.kernel,
    mesh=_sc_mesh,
    out_type=jax.ShapeDtypeStruct((_S, _D), jnp.float32),
    scratch_types=[
        pltpu.VMEM((_SPW,), jnp.int32),
        pltpu.VMEM((_SPW, _D), jnp.float32),
        pltpu.SemaphoreType.DMA,
    ],
)
def _sc_gather_samples(weight_hbm, sids_hbm, out_s, sidx_v, srows_v, sem):
    wid = lax.axis_index("s") * _NC + lax.axis_index("c")
    sbase = wid * _SPW
    pltpu.sync_copy(sids_hbm.at[pl.ds(sbase, _SPW)], sidx_v)
    pltpu.async_copy(weight_hbm.at[sidx_v], srows_v, sem).wait()
    pltpu.sync_copy(srows_v, out_s.at[pl.ds(sbase, _SPW)])


@functools.partial(
    pl.kernel,
    mesh=_sc_mesh,
    out_type=jax.ShapeDtypeStruct((_NW, _L), jnp.float32),
    scratch_types=[
        pltpu.VMEM((_BPW,), jnp.int32),
        pltpu.VMEM((_BPW, _D), jnp.float32),
        pltpu.VMEM((_BPW, _D), jnp.float32),
        pltpu.VMEM((_L,), jnp.float32),
        pltpu.SemaphoreType.DMA,
    ],
)
def _sc_true_dot(x_hbm, labels_hbm, weight_hbm, out_p,
                 lidx_v, lrows_v, xrows_v, acc_v, sem):
    wid = lax.axis_index("s") * _NC + lax.axis_index("c")
    lbase = wid * _BPW
    pltpu.sync_copy(labels_hbm.at[pl.ds(lbase, _BPW)], lidx_v)
    cp_l = pltpu.async_copy(weight_hbm.at[lidx_v], lrows_v, sem)
    pltpu.sync_copy(x_hbm.at[pl.ds(lbase, _BPW)], xrows_v)
    cp_l.wait()

    def body(r, acc):
        for c in range(_D // _L):
            acc = acc + (lrows_v[r, pl.ds(c * _L, _L)]
                         * xrows_v[r, pl.ds(c * _L, _L)])
        return acc

    acc_v[...] = lax.fori_loop(0, _BPW, body, jnp.zeros((_L,), jnp.float32))
    pltpu.sync_copy(acc_v, out_p.at[wid])


def _lse_body(x_ref, sw_ref, out_ref):
    i = pl.program_id(0)
    logits = lax.dot_general(
        x_ref[...], sw_ref[...], (((1,), (1,)), ((), ())),
        preferred_element_type=jnp.float32)
    rowsum = jnp.sum(jnp.exp(logits), axis=1)
    contrib = jnp.sum(jnp.log(rowsum))

    @pl.when(i == 0)
    def _():
        out_ref[0, 0] = contrib

    @pl.when(i != 0)
    def _():
        out_ref[0, 0] += contrib


def _tc_lse(x, sw):
    out = pl.pallas_call(
        _lse_body,
        grid=(_B // _BT,),
        in_specs=[
            pl.BlockSpec((_BT, _D), lambda i: (i, 0)),
            pl.BlockSpec((_S, _D), lambda i: (0, 0)),
        ],
        out_specs=pl.BlockSpec((1, 1), lambda i: (0, 0),
                               memory_space=pltpu.SMEM),
        out_shape=jax.ShapeDtypeStruct((1, 1), jnp.float32),
        cost_estimate=pl.CostEstimate(
            flops=2 * _B * _S * _D, transcendentals=_B * _S,
            bytes_accessed=(_B * _D * 4 + _S * _D * 4)),
    )(x, sw)
    return out[0, 0]


def kernel(inputs, labels, sample_ids, weight):
    sw = _sc_gather_samples(weight, sample_ids.astype(jnp.int32))
    lse = _tc_lse(inputs, sw)
    part = _sc_true_dot(inputs, labels.astype(jnp.int32), weight)
    return lse - jnp.sum(part)


# speedup vs baseline: 1.1272x; 1.0073x over previous
import functools

import jax
import jax.numpy as jnp
from jax import lax
from jax.experimental import pallas as pl
from jax.experimental.pallas import tpu as pltpu
from jax.experimental.pallas import tpu_sc as plsc

_B = 4096
_S = 8192
_D = 128
_BT = 2048
_L = 16

_info = plsc.get_sparse_core_info()
_NC = _info.num_cores
_NS = _info.num_subcores
_NW = _NC * _NS
_SPW = _S // _NW
_BPW = _B // _NW

_sc_mesh = plsc.VectorSubcoreMesh(core_axis_name="c", subcore_axis_name="s")


@functools.partial(
    pl.kernel,
    mesh=_sc_mesh,
    out_type=jax.ShapeDtypeStruct((_S, _D), jnp.float32),
    scratch_types=[
        pltpu.VMEM((_SPW,), jnp.int32),
        pltpu.VMEM((_SPW // 2, _D), jnp.float32),
        pltpu.VMEM((_SPW // 2, _D), jnp.float32),
        pltpu.SemaphoreType.DMA,
        pltpu.SemaphoreType.DMA,
        pltpu.SemaphoreType.DMA,
        pltpu.SemaphoreType.DMA,
    ],
)
def _sc_gather_samples(weight_hbm, sids_hbm, out_s, sidx_v, rows_a, rows_b,
                       sem_a, sem_b, sem_wa, sem_wb):
    wid = lax.axis_index("s") * _NC + lax.axis_index("c")
    half = _SPW // 2
    sbase = wid * _SPW
    pltpu.sync_copy(sids_hbm.at[pl.ds(sbase, _SPW)], sidx_v)
    cp_a = pltpu.async_copy(weight_hbm.at[sidx_v.at[pl.ds(0, half)]],
                            rows_a, sem_a)
    cp_b = pltpu.async_copy(weight_hbm.at[sidx_v.at[pl.ds(half, half)]],
                            rows_b, sem_b)
    cp_a.wait()
    wb_a = pltpu.async_copy(rows_a, out_s.at[pl.ds(sbase, half)], sem_wa)
    cp_b.wait()
    wb_b = pltpu.async_copy(rows_b, out_s.at[pl.ds(sbase + half, half)],
                            sem_wb)
    wb_a.wait()
    wb_b.wait()


@functools.partial(
    pl.kernel,
    mesh=_sc_mesh,
    out_type=jax.ShapeDtypeStruct((_NW, _L), jnp.float32),
    scratch_types=[
        pltpu.VMEM((_BPW,), jnp.int32),
        pltpu.VMEM((_BPW, _D), jnp.float32),
        pltpu.VMEM((_BPW, _D), jnp.float32),
        pltpu.VMEM((_L,), jnp.float32),
        pltpu.SemaphoreType.DMA,
    ],
)
def _sc_true_dot(x_hbm, labels_hbm, weight_hbm, out_p,
                 lidx_v, lrows_v, xrows_v, acc_v, sem):
    wid = lax.axis_index("s") * _NC + lax.axis_index("c")
    lbase = wid * _BPW
    pltpu.sync_copy(labels_hbm.at[pl.ds(lbase, _BPW)], lidx_v)
    cp_l = pltpu.async_copy(weight_hbm.at[lidx_v], lrows_v, sem)
    pltpu.sync_copy(x_hbm.at[pl.ds(lbase, _BPW)], xrows_v)
    cp_l.wait()

    def body(r, acc):
        for c in range(_D // _L):
            acc = acc + (lrows_v[r, pl.ds(c * _L, _L)]
                         * xrows_v[r, pl.ds(c * _L, _L)])
        return acc

    acc_v[...] = lax.fori_loop(0, _BPW, body, jnp.zeros((_L,), jnp.float32))
    pltpu.sync_copy(acc_v, out_p.at[wid])


def _lse_body(x_ref, sw_ref, out_ref):
    i = pl.program_id(0)
    logits = lax.dot_general(
        x_ref[...].astype(jnp.bfloat16), sw_ref[...].astype(jnp.bfloat16),
        (((1,), (1,)), ((), ())),
        preferred_element_type=jnp.float32)
    rowsum = jnp.sum(jnp.exp(logits), axis=1)
    contrib = jnp.sum(jnp.log(rowsum))

    @pl.when(i == 0)
    def _():
        out_ref[0, 0] = contrib

    @pl.when(i != 0)
    def _():
        out_ref[0, 0] += contrib


def _tc_lse(x, sw):
    out = pl.pallas_call(
        _lse_body,
        grid=(_B // _BT,),
        in_specs=[
            pl.BlockSpec((_BT, _D), lambda i: (i, 0)),
            pl.BlockSpec((_S, _D), lambda i: (0, 0)),
        ],
        out_specs=pl.BlockSpec((1, 1), lambda i: (0, 0),
                               memory_space=pltpu.SMEM),
        out_shape=jax.ShapeDtypeStruct((1, 1), jnp.float32),
        cost_estimate=pl.CostEstimate(
            flops=2 * _B * _S * _D, transcendentals=_B * _S,
            bytes_accessed=(_B * _D * 4 + _S * _D * 4)),
    )(x, sw)
    return out[0, 0]


def kernel(inputs, labels, sample_ids, weight):
    sw = _sc_gather_samples(weight, sample_ids.astype(jnp.int32))
    lse = _tc_lse(inputs, sw)
    part = _sc_true_dot(inputs, labels.astype(jnp.int32), weight)
    return lse - jnp.sum(part)
